# baseline (speedup 1.0000x reference)
import functools

import jax
import jax.numpy as jnp
import numpy as np
from jax.experimental import pallas as pl
from jax.experimental.pallas import tpu as pltpu

K = 1024
CD = 10
D = 64
SCALE = 1.0
INV_TEMP = 100.0
ENT_W = 0.1
COMMIT_W = 0.25
GAMMA = 1.0
B, N = 8, 4096
TOKENS = B * N
TILE = 512
GRID = TOKENS // TILE

_mask = 2 ** np.arange(CD - 1, -1, -1)
_bits = ((np.arange(K)[:, None] & _mask) != 0).astype(np.float32)
_CODEBOOK = _bits * SCALE * 2.0 - SCALE
_CT200 = (2.0 * INV_TEMP * _CODEBOOK.T).astype(np.float32)
_IMASK = _mask.astype(np.int32)


def _lfq_body(z_ref, wi_ref, bi_ref, wo_ref, bo_ref, ct_ref,
              out_ref, idx_ref, aux_ref,
              avg_acc, sums_acc):
    step = pl.program_id(0)

    @pl.when(step == 0)
    def _init():
        avg_acc[...] = jnp.zeros_like(avg_acc)
        sums_acc[0] = 0.0
        sums_acc[1] = 0.0

    z = z_ref[...]
    x = jax.lax.dot_general(z, wi_ref[...], (((1,), (1,)), ((), ())),
                            preferred_element_type=jnp.float32) + bi_ref[...]
    pos = x > 0
    q = jnp.where(pos, SCALE, -SCALE).astype(jnp.float32)

    j = jax.lax.broadcasted_iota(jnp.int32, (1, CD), 1)
    imask = jnp.left_shift(1, CD - 1 - j)
    idx = jnp.sum(pos.astype(jnp.int32) * imask, axis=1, keepdims=True)
    idx_ref[...] = idx

    out_ref[...] = jax.lax.dot_general(q, wo_ref[...], (((1,), (1,)), ((), ())),
                                       preferred_element_type=jnp.float32) + bo_ref[...]

    commit_tile = jnp.sum((x - q) ** 2)

    l = jax.lax.dot_general(x, ct_ref[...], (((1,), (0,)), ((), ())),
                            preferred_element_type=jnp.float32)
    m = jnp.max(l, axis=1, keepdims=True)
    lp = l - m
    e = jnp.exp(lp)
    s = jnp.sum(e, axis=1, keepdims=True)
    r = 1.0 / s
    w = jnp.sum(e * lp, axis=1, keepdims=True)
    ent_tile = jnp.sum(jnp.log(s) - w * r)
    p = e * r
    avg_acc[...] += jnp.sum(p, axis=0, keepdims=True)

    sums_acc[0] += ent_tile
    sums_acc[1] += commit_tile

    @pl.when(step == GRID - 1)
    def _fin():
        nt = float(TOKENS)
        pse = sums_acc[0] / nt
        ap = avg_acc[...] / nt
        ce = jnp.sum(-ap * jnp.log(jnp.clip(ap, 1e-20, None)))
        commit = sums_acc[1] / (nt * CD)
        aux = (pse - GAMMA * ce) * ENT_W + COMMIT_W * commit
        aux_ref[...] = jnp.reshape(aux, (1, 1))


@functools.partial(jax.jit, static_argnames=())
def kernel(z_e_x, W_in, b_in, W_out, b_out):
    z2 = z_e_x.reshape(TOKENS, D)
    bi = b_in.reshape(1, CD)
    bo = b_out.reshape(1, D)
    ct = jnp.asarray(_CT200)

    out2, idx2, aux = pl.pallas_call(
        _lfq_body,
        grid=(GRID,),
        in_specs=[
            pl.BlockSpec((TILE, D), lambda i: (i, 0)),
            pl.BlockSpec((CD, D), lambda i: (0, 0)),
            pl.BlockSpec((1, CD), lambda i: (0, 0)),
            pl.BlockSpec((D, CD), lambda i: (0, 0)),
            pl.BlockSpec((1, D), lambda i: (0, 0)),
            pl.BlockSpec((CD, K), lambda i: (0, 0)),
        ],
        out_specs=[
            pl.BlockSpec((TILE, D), lambda i: (i, 0)),
            pl.BlockSpec((TILE, 1), lambda i: (i, 0)),
            pl.BlockSpec((1, 1), lambda i: (0, 0)),
        ],
        out_shape=[
            jax.ShapeDtypeStruct((TOKENS, D), jnp.float32),
            jax.ShapeDtypeStruct((TOKENS, 1), jnp.int32),
            jax.ShapeDtypeStruct((1, 1), jnp.float32),
        ],
        scratch_shapes=[
            pltpu.VMEM((1, K), jnp.float32),
            pltpu.SMEM((2,), jnp.float32),
        ],
    )(z2, W_in, bi, W_out, bo, ct)

    out = out2.reshape(B, N, D)
    indices = idx2.reshape(B, N)
    aux_loss = aux.reshape(())
    return (out, indices, aux_loss)


# speedup vs baseline: 1.1581x; 1.1581x over previous
import functools

import jax
import jax.numpy as jnp
import numpy as np
from jax.experimental import pallas as pl
from jax.experimental.pallas import tpu as pltpu

K = 1024
CD = 10
D = 64
SCALE = 1.0
INV_TEMP = 100.0
ENT_W = 0.1
COMMIT_W = 0.25
GAMMA = 1.0
B, N = 8, 4096
TOKENS = B * N
TILE = 512
GRID = TOKENS // TILE

_s = 2.0 * INV_TEMP * SCALE
_CT73 = np.zeros((CD, 136), dtype=np.float32)
for _d in range(7):
    _J = np.arange(128)
    _CT73[_d, :128] = _s * (2.0 * ((_J >> (6 - _d)) & 1) - 1.0)
for _d in range(7, CD):
    _L = np.arange(8)
    _CT73[_d, 128:136] = _s * (2.0 * ((_L >> (9 - _d)) & 1) - 1.0)


def _softmax_stats(l):
    m = jnp.max(l, axis=1, keepdims=True)
    lp = l - m
    e = jnp.exp(lp)
    s = jnp.sum(e, axis=1, keepdims=True)
    r = 1.0 / s
    w = jnp.sum(e * lp, axis=1, keepdims=True)
    return e * r, jnp.log(s) - w * r


def _lfq_body(z_ref, wi_ref, bi_ref, wo_ref, bo_ref, ct_ref,
              out_ref, idx_ref, aux_ref,
              avg_acc, sums_acc):
    step = pl.program_id(0)

    @pl.when(step == 0)
    def _init():
        avg_acc[...] = jnp.zeros_like(avg_acc)
        sums_acc[0] = 0.0
        sums_acc[1] = 0.0

    z = z_ref[...]
    x = jax.lax.dot_general(z, wi_ref[...], (((1,), (1,)), ((), ())),
                            preferred_element_type=jnp.float32) + bi_ref[...]
    pos = x > 0
    q = jnp.where(pos, SCALE, -SCALE).astype(jnp.float32)

    j = jax.lax.broadcasted_iota(jnp.int32, (1, CD), 1)
    imask = jnp.left_shift(1, CD - 1 - j)
    idx = jnp.sum(pos.astype(jnp.int32) * imask, axis=1, keepdims=True)
    idx_ref[...] = idx

    out_ref[...] = jax.lax.dot_general(q, wo_ref[...], (((1,), (1,)), ((), ())),
                                       preferred_element_type=jnp.float32) + bo_ref[...]

    commit_tile = jnp.sum((x - q) ** 2)

    y = jax.lax.dot_general(x, ct_ref[...], (((1,), (0,)), ((), ())),
                            preferred_element_type=jnp.float32)
    p7, h7 = _softmax_stats(y[:, :128])
    p3, h3 = _softmax_stats(y[:, 128:136])
    ent_tile = jnp.sum(h7) + jnp.sum(h3)

    avg_acc[...] += jax.lax.dot_general(p7, p3, (((0,), (0,)), ((), ())),
                                        preferred_element_type=jnp.float32)

    sums_acc[0] += ent_tile
    sums_acc[1] += commit_tile

    @pl.when(step == GRID - 1)
    def _fin():
        nt = float(TOKENS)
        pse = sums_acc[0] / nt
        ap = avg_acc[...] / nt
        ce = jnp.sum(-ap * jnp.log(jnp.clip(ap, 1e-20, None)))
        commit = sums_acc[1] / (nt * CD)
        aux = (pse - GAMMA * ce) * ENT_W + COMMIT_W * commit
        aux_ref[...] = jnp.reshape(aux, (1, 1))


@functools.partial(jax.jit, static_argnames=())
def kernel(z_e_x, W_in, b_in, W_out, b_out):
    z2 = z_e_x.reshape(TOKENS, D)
    bi = b_in.reshape(1, CD)
    bo = b_out.reshape(1, D)
    ct = jnp.asarray(_CT73)

    out2, idx2, aux = pl.pallas_call(
        _lfq_body,
        grid=(GRID,),
        in_specs=[
            pl.BlockSpec((TILE, D), lambda i: (i, 0)),
            pl.BlockSpec((CD, D), lambda i: (0, 0)),
            pl.BlockSpec((1, CD), lambda i: (0, 0)),
            pl.BlockSpec((D, CD), lambda i: (0, 0)),
            pl.BlockSpec((1, D), lambda i: (0, 0)),
            pl.BlockSpec((CD, 136), lambda i: (0, 0)),
        ],
        out_specs=[
            pl.BlockSpec((TILE, D), lambda i: (i, 0)),
            pl.BlockSpec((TILE, 1), lambda i: (i, 0)),
            pl.BlockSpec((1, 1), lambda i: (0, 0)),
        ],
        out_shape=[
            jax.ShapeDtypeStruct((TOKENS, D), jnp.float32),
            jax.ShapeDtypeStruct((TOKENS, 1), jnp.int32),
            jax.ShapeDtypeStruct((1, 1), jnp.float32),
        ],
        scratch_shapes=[
            pltpu.VMEM((128, 8), jnp.float32),
            pltpu.SMEM((2,), jnp.float32),
        ],
    )(z2, W_in, bi, W_out, bo, ct)

    out = out2.reshape(B, N, D)
    indices = idx2.reshape(B, N)
    aux_loss = aux.reshape(())
    return (out, indices, aux_loss)
